# Initial kernel scaffold; baseline (speedup 1.0000x reference)
import jax
import jax.numpy as jnp
from jax.experimental import pallas as pl


def kernel(seqs, seg_label, token_table, seg_table, pe):
    raise NotImplementedError("write your pallas kernel here")


# speedup vs baseline: 2.8408x; 2.8408x over previous
import functools

import jax
import jax.numpy as jnp
from jax import lax
from jax.experimental import pallas as pl
from jax.experimental.pallas import tpu as pltpu
from jax.experimental.pallas import tpu_sc as plsc

D = 128
L_SEQ = 200
C = 128


def _make_sc_kernel(n_tok):
  info = plsc.get_sparse_core_info()
  nc, ns = info.num_cores, info.num_subcores
  nw = nc * ns
  per_w = n_tok // nw
  n_chunks = per_w // C
  assert per_w * nw == n_tok and n_chunks * C == per_w

  mesh = plsc.VectorSubcoreMesh(core_axis_name="c", subcore_axis_name="s")

  @functools.partial(
      pl.kernel,
      mesh=mesh,
      compiler_params=pltpu.CompilerParams(needs_layout_passes=False),
      out_type=jax.ShapeDtypeStruct((n_tok, D), jnp.float32),
      scratch_types=[
          pltpu.VMEM((L_SEQ, D), jnp.float32),
          pltpu.VMEM((3 * D,), jnp.float32),
          pltpu.VMEM((C,), jnp.int32),
          pltpu.VMEM((C,), jnp.int32),
          pltpu.VMEM((C, D), jnp.float32),
          pltpu.SemaphoreType.DMA,
      ],
  )
  def sc_embed(seqs_hbm, segl_hbm, tbl_hbm, segtab_hbm, pe_hbm, out_hbm,
               pe_v, segtab_v, idx_v, segl_v, rows_v, sem):
    wid = lax.axis_index("s") * nc + lax.axis_index("c")
    pltpu.sync_copy(pe_hbm, pe_v)
    pltpu.sync_copy(segtab_hbm, segtab_v)
    lane = lax.iota(jnp.int32, 16)

    def chunk_body(t, carry):
      base = wid * per_w + t * C
      pltpu.sync_copy(seqs_hbm.at[pl.ds(base, C)], idx_v)
      pltpu.sync_copy(segl_hbm.at[pl.ds(base, C)], segl_v)
      pltpu.async_copy(tbl_hbm.at[idx_v], rows_v, sem).wait()

      def row_body(r, rcarry):
        l = lax.rem(base + r, L_SEQ)
        s_splat = plsc.load_gather(segl_v, [jnp.full((16,), r, jnp.int32)])
        sbase = s_splat * D + lane
        for c in range(D // 16):
          sl = pl.ds(c * 16, 16)
          seg_vals = plsc.load_gather(segtab_v, [sbase + c * 16])
          rows_v[r, sl] = rows_v[r, sl] + pe_v[l, sl] + seg_vals
        return rcarry

      lax.fori_loop(0, C, row_body, 0)
      pltpu.sync_copy(rows_v, out_hbm.at[pl.ds(base, C)])
      return carry

    lax.fori_loop(0, n_chunks, chunk_body, 0)

  return sc_embed


@jax.jit
def kernel(seqs, seg_label, token_table, seg_table, pe):
  b, l = seqs.shape
  n_tok = b * l
  seqs_f = seqs.reshape(n_tok).astype(jnp.int32)
  segl_f = seg_label.reshape(n_tok).astype(jnp.int32)
  pe2 = pe.reshape(pe.shape[1], pe.shape[2])[:l]
  out = _make_sc_kernel(n_tok)(seqs_f, segl_f, token_table,
                               seg_table.reshape(-1), pe2)
  return out.reshape(b, l, D)


# speedup vs baseline: 4.3280x; 1.5235x over previous
import functools

import jax
import jax.numpy as jnp
from jax import lax
from jax.experimental import pallas as pl
from jax.experimental.pallas import tpu as pltpu
from jax.experimental.pallas import tpu_sc as plsc

D = 128
L_SEQ = 200
C = 64
NBUF = 4


def _make_sc_kernel(n_tok):
  info = plsc.get_sparse_core_info()
  nc, ns = info.num_cores, info.num_subcores
  nw = nc * ns
  per_w = n_tok // nw
  n_chunks = per_w // C
  assert per_w * nw == n_tok and n_chunks * C == per_w
  assert n_chunks % NBUF == 0 and n_chunks >= 2 * NBUF

  mesh = plsc.VectorSubcoreMesh(core_axis_name="c", subcore_axis_name="s")

  @functools.partial(
      pl.kernel,
      mesh=mesh,
      compiler_params=pltpu.CompilerParams(needs_layout_passes=False),
      out_type=jax.ShapeDtypeStruct((n_tok, D), jnp.float32),
      scratch_types=[
          pltpu.VMEM((L_SEQ, D), jnp.float32),
          pltpu.VMEM((3 * D,), jnp.float32),
          pltpu.VMEM((per_w,), jnp.int32),
          pltpu.VMEM((per_w,), jnp.int32),
          *([pltpu.VMEM((C, D), jnp.float32)] * NBUF),
          *([pltpu.SemaphoreType.DMA] * NBUF),
          *([pltpu.SemaphoreType.DMA] * NBUF),
      ],
  )
  def sc_embed(seqs_hbm, segl_hbm, tbl_hbm, segtab_hbm, pe_hbm, out_hbm,
               pe_v, segtab_v, idx_v, segl_v, *bufs_and_sems):
    rows = bufs_and_sems[:NBUF]
    gsem = bufs_and_sems[NBUF:2 * NBUF]
    osem = bufs_and_sems[2 * NBUF:3 * NBUF]

    wid = lax.axis_index("s") * nc + lax.axis_index("c")
    wbase = wid * per_w
    pltpu.sync_copy(seqs_hbm.at[pl.ds(wbase, per_w)], idx_v)
    pltpu.sync_copy(segl_hbm.at[pl.ds(wbase, per_w)], segl_v)
    pltpu.sync_copy(pe_hbm, pe_v)
    pltpu.sync_copy(segtab_hbm, segtab_v)
    seg_regs = [[segtab_v[pl.ds(s * D + c * 16, 16)] for c in range(D // 16)]
                for s in range(3)]

    def g_copy(t, b):
      return pltpu.make_async_copy(
          tbl_hbm.at[idx_v.at[pl.ds(t * C, C)]], rows[b], gsem[b])

    def o_copy(t, b):
      return pltpu.make_async_copy(
          rows[b], out_hbm.at[pl.ds(wbase + t * C, C)], osem[b])

    g_copy(0, 0).start()
    g_copy(1, 1).start()

    def compute(t, b):
      rbase = wbase + t * C
      rref = rows[b]

      def row_body(r, carry):
        l = lax.rem(rbase + r, L_SEQ)
        sv = plsc.load_gather(segl_v, [jnp.full((16,), t * C + r, jnp.int32)])
        m1 = sv == 1
        m2 = sv == 2
        for c in range(D // 16):
          sl = pl.ds(c * 16, 16)
          seg = jnp.where(m1, seg_regs[1][c],
                          jnp.where(m2, seg_regs[2][c], seg_regs[0][c]))
          rref[r, sl] = rref[r, sl] + pe_v[l, sl] + seg
        return carry

      lax.fori_loop(0, C, row_body, 0, unroll=2)

    def outer(T, carry):
      for j in range(NBUF):
        t = T + j
        bn = (j + 2) % NBUF

        @pl.when(t + 2 < n_chunks)
        def _prefetch():
          @pl.when(t >= 2)
          def _drain():
            o_copy(t - 2, bn).wait()
          g_copy(t + 2, bn).start()

        g_copy(t, j).wait()
        compute(t, j)
        o_copy(t, j).start()
      return carry

    lax.fori_loop(0, n_chunks // NBUF, lambda i, c: outer(i * NBUF, c), 0)
    for j in range(NBUF):
      o_copy(n_chunks - NBUF + j, j).wait()

  return sc_embed


@jax.jit
def kernel(seqs, seg_label, token_table, seg_table, pe):
  b, l = seqs.shape
  n_tok = b * l
  seqs_f = seqs.reshape(n_tok).astype(jnp.int32)
  segl_f = seg_label.reshape(n_tok).astype(jnp.int32)
  pe2 = pe.reshape(pe.shape[1], pe.shape[2])[:l]
  out = _make_sc_kernel(n_tok)(seqs_f, segl_f, token_table,
                               seg_table.reshape(-1), pe2)
  return out.reshape(b, l, D)
